# Initial kernel scaffold; baseline (speedup 1.0000x reference)
import jax
import jax.numpy as jnp
from jax.experimental import pallas as pl


def kernel(x, edge_index, batch, W1_rel, b1_rel, W1_root, pw1, W2_rel, b2_rel, W2_root, pw2, W_l1, b_l1, W_l2, b_l2, W_l3, b_l3):
    raise NotImplementedError("write your pallas kernel here")


# speedup vs baseline: 1.0045x; 1.0045x over previous
import functools
import jax
import jax.numpy as jnp
from jax.experimental import pallas as pl
from jax.experimental.pallas import tpu as pltpu

NUM_GRAPHS = 128
RATIO = 0.8


def _head_body(z_ref, w1_ref, b1_ref, w2_ref, b2_ref, w3_ref, b3_ref, o_ref):
    z = z_ref[...]
    h = jnp.maximum(jnp.dot(z, w1_ref[...], preferred_element_type=jnp.float32) + b1_ref[...], 0.0)
    h = jnp.maximum(jnp.dot(h, w2_ref[...], preferred_element_type=jnp.float32) + b2_ref[...], 0.0)
    o = jnp.dot(h, w3_ref[...], preferred_element_type=jnp.float32) + b3_ref[...]
    m = jnp.max(o, axis=-1, keepdims=True)
    lse = m + jnp.log(jnp.sum(jnp.exp(o - m), axis=-1, keepdims=True))
    o_ref[...] = o - lse


def _mlp_head(z, W_l1, b_l1, W_l2, b_l2, W_l3, b_l3):
    return pl.pallas_call(
        _head_body,
        out_shape=jax.ShapeDtypeStruct((z.shape[0], W_l3.shape[0]), jnp.float32),
    )(z, W_l1.T, b_l1[None, :], W_l2.T, b_l2[None, :], W_l3.T, b_l3[None, :])


def _graph_conv(x, src, dst, W_rel, b_rel, W_root, emask=None):
    msg = x[src]
    if emask is not None:
        msg = msg * emask[:, None]
    agg = jnp.zeros_like(x).at[dst].add(msg)
    return agg @ W_rel.T + b_rel[None, :] + x @ W_root.T


def _topk_select(score, batch, node_mask, k_per_graph, num_graphs):
    s = jnp.where(node_mask, score, -jnp.inf)
    sort_key = batch.astype(jnp.float32) * 4.0 - s
    order = jnp.argsort(sort_key)
    active = jax.ops.segment_sum(node_mask.astype(jnp.int32), batch, num_segments=num_graphs)
    starts = jnp.cumsum(active) - active
    n = score.shape[0]
    rank = jnp.zeros((n,), jnp.int32).at[order].set(jnp.arange(n, dtype=jnp.int32) - starts[batch[order]])
    return (rank < k_per_graph[batch]) & node_mask


def _global_pools(x, batch, sel, cnt, num_graphs):
    mx = jax.ops.segment_max(jnp.where(sel[:, None], x, -jnp.inf), batch, num_segments=num_graphs)
    mx = jnp.where(jnp.isfinite(mx), mx, 0.0)
    sm = jax.ops.segment_sum(jnp.where(sel[:, None], x, 0.0), batch, num_segments=num_graphs)
    mean = sm / jnp.maximum(cnt, 1).astype(jnp.float32)[:, None]
    return jnp.concatenate([mx, mean], axis=1)


def kernel(x, edge_index, batch, W1_rel, b1_rel, W1_root, pw1, W2_rel, b2_rel, W2_root, pw2, W_l1, b_l1, W_l2, b_l2, W_l3, b_l3):
    src, dst = edge_index[0], edge_index[1]
    G = NUM_GRAPHS
    counts = jax.ops.segment_sum(jnp.ones_like(batch), batch, num_segments=G)
    k1 = jnp.ceil(RATIO * counts.astype(jnp.float32)).astype(jnp.int32)
    k2 = jnp.ceil(RATIO * k1.astype(jnp.float32)).astype(jnp.int32)

    h = jax.nn.relu(_graph_conv(x, src, dst, W1_rel, b1_rel, W1_root))
    s1 = jnp.tanh((h @ pw1) / jnp.linalg.norm(pw1))
    all_mask = jnp.ones((h.shape[0],), dtype=bool)
    sel1 = _topk_select(s1, batch, all_mask, k1, G)
    hp1 = h * s1[:, None] * sel1[:, None].astype(h.dtype)
    x1 = _global_pools(hp1, batch, sel1, k1, G)

    emask = (sel1[src] & sel1[dst]).astype(hp1.dtype)
    h2 = jax.nn.relu(_graph_conv(hp1, src, dst, W2_rel, b2_rel, W2_root, emask))
    s2 = jnp.tanh((h2 @ pw2) / jnp.linalg.norm(pw2))
    sel2 = _topk_select(s2, batch, sel1, k2, G)
    hp2 = h2 * s2[:, None] * sel2[:, None].astype(h2.dtype)
    x2 = _global_pools(hp2, batch, sel2, k2, G)

    z = x1 + x2
    return _mlp_head(z, W_l1, b_l1, W_l2, b_l2, W_l3, b_l3)


# speedup vs baseline: 22.4461x; 22.3446x over previous
import functools
import jax
import jax.numpy as jnp
from jax import lax
from jax.experimental import pallas as pl
from jax.experimental.pallas import tpu as pltpu
from jax.experimental.pallas import tpu_sc as plsc

NUM_GRAPHS = 128
RATIO = 0.8

N_NODES = 100000
N_EDGES = 3200000
NC = 2
NS = 16
EDGES_PER_WORKER = N_EDGES // (NC * NS)
ROWS_PER_TILE = 6256
N_PAD = ROWS_PER_TILE * NS


def _edge_win(d):
    return 1000 if d else 2000


ZCHUNK = 272
N_ZCOPY = ROWS_PER_TILE // ZCHUNK


def _make_edge_agg_body(d):
    zrows = ZCHUNK * (d if d else 1) // 16

    ew = _edge_win(d)
    n_wins = EDGES_PER_WORKER // ew

    def body(table_hbm, src_hbm, dst_hbm, out_hbm,
             agg_sh, src_v, dst_v, rows_v, zbuf, gsem):
        c = lax.axis_index("c")
        s = lax.axis_index("s")
        wid = c * NS + s

        def zstore(i, carry):
            if d:
                zbuf[i, :] = jnp.zeros((16,), jnp.float32)
            else:
                zbuf[pl.ds(i * 16, 16)] = jnp.zeros((16,), jnp.float32)
            return carry

        lax.fori_loop(0, (ZCHUNK * d // 16) if d else (ZCHUNK // 16), zstore, 0)
        for j in range(N_ZCOPY):
            pltpu.sync_copy(
                zbuf, agg_sh.at[pl.ds(s * ROWS_PER_TILE + j * ZCHUNK, ZCHUNK)])
        plsc.subcore_barrier()

        def win(w, carry):
            base = wid * EDGES_PER_WORKER + w * ew
            pltpu.sync_copy(src_hbm.at[pl.ds(base, ew)], src_v)
            pltpu.async_copy(table_hbm.at[src_v], rows_v, gsem).wait()
            pltpu.sync_copy(dst_hbm.at[pl.ds(base, ew)], dst_v)
            pltpu.sync_copy(rows_v, agg_sh.at[dst_v], add=True)
            return carry

        lax.fori_loop(0, n_wins, win, 0)
        plsc.subcore_barrier()
        for j in range(N_ZCOPY):
            off = s * ROWS_PER_TILE + j * ZCHUNK
            pltpu.sync_copy(agg_sh.at[pl.ds(off, ZCHUNK)], zbuf)
            pltpu.sync_copy(zbuf, out_hbm.at[pl.ds(c * N_PAD + off, ZCHUNK)])

    return body


def _edge_agg_rows(table, src, dst, d):
    mesh = plsc.VectorSubcoreMesh(core_axis_name="c", subcore_axis_name="s")
    if d:
        row = (d,)
    else:
        row = ()
    fn = pl.kernel(
        _make_edge_agg_body(d),
        out_type=jax.ShapeDtypeStruct((NC * N_PAD,) + row, jnp.float32),
        mesh=mesh,
        compiler_params=pltpu.CompilerParams(use_tc_tiling_on_sc=False),
        scratch_types=[
            pltpu.VMEM_SHARED((N_PAD,) + row, jnp.float32),
            pltpu.VMEM((_edge_win(d),), jnp.int32),
            pltpu.VMEM((_edge_win(d),), jnp.int32),
            pltpu.VMEM((_edge_win(d),) + row, jnp.float32),
            pltpu.VMEM((ZCHUNK,) + row, jnp.float32),
            pltpu.SemaphoreType.DMA,
        ],
    )
    parts = fn(table, src, dst)
    return parts[:N_NODES] + parts[N_PAD:N_PAD + N_NODES]


def _head_body(z_ref, w1_ref, b1_ref, w2_ref, b2_ref, w3_ref, b3_ref, o_ref):
    z = z_ref[...]
    h = jnp.maximum(jnp.dot(z, w1_ref[...], preferred_element_type=jnp.float32) + b1_ref[...], 0.0)
    h = jnp.maximum(jnp.dot(h, w2_ref[...], preferred_element_type=jnp.float32) + b2_ref[...], 0.0)
    o = jnp.dot(h, w3_ref[...], preferred_element_type=jnp.float32) + b3_ref[...]
    m = jnp.max(o, axis=-1, keepdims=True)
    lse = m + jnp.log(jnp.sum(jnp.exp(o - m), axis=-1, keepdims=True))
    o_ref[...] = o - lse


def _mlp_head(z, W_l1, b_l1, W_l2, b_l2, W_l3, b_l3):
    return pl.pallas_call(
        _head_body,
        out_shape=jax.ShapeDtypeStruct((z.shape[0], W_l3.shape[0]), jnp.float32),
    )(z, W_l1.T, b_l1[None, :], W_l2.T, b_l2[None, :], W_l3.T, b_l3[None, :])


def _graph_conv1(x, src, dst, W_rel, b_rel, W_root):
    agg = _edge_agg_rows(x[:, 0], src, dst, 0)[:, None]
    return agg @ W_rel.T + b_rel[None, :] + x @ W_root.T


def _graph_conv2(hp1, sel1, src, dst, W_rel, b_rel, W_root):
    agg = _edge_agg_rows(hp1, src, dst, 16)
    agg = agg * sel1[:, None].astype(hp1.dtype)
    return agg @ W_rel.T + b_rel[None, :] + hp1 @ W_root.T


def _topk_select(score, batch, node_mask, k_per_graph, num_graphs):
    s = jnp.where(node_mask, score, -jnp.inf)
    sort_key = batch.astype(jnp.float32) * 4.0 - s
    order = jnp.argsort(sort_key)
    active = jax.ops.segment_sum(node_mask.astype(jnp.int32), batch, num_segments=num_graphs)
    starts = jnp.cumsum(active) - active
    n = score.shape[0]
    rank = jnp.zeros((n,), jnp.int32).at[order].set(jnp.arange(n, dtype=jnp.int32) - starts[batch[order]])
    return (rank < k_per_graph[batch]) & node_mask


def _global_pools(x, batch, sel, cnt, num_graphs):
    mx = jax.ops.segment_max(jnp.where(sel[:, None], x, -jnp.inf), batch, num_segments=num_graphs)
    mx = jnp.where(jnp.isfinite(mx), mx, 0.0)
    sm = jax.ops.segment_sum(jnp.where(sel[:, None], x, 0.0), batch, num_segments=num_graphs)
    mean = sm / jnp.maximum(cnt, 1).astype(jnp.float32)[:, None]
    return jnp.concatenate([mx, mean], axis=1)


def kernel(x, edge_index, batch, W1_rel, b1_rel, W1_root, pw1, W2_rel, b2_rel, W2_root, pw2, W_l1, b_l1, W_l2, b_l2, W_l3, b_l3):
    src, dst = edge_index[0], edge_index[1]
    G = NUM_GRAPHS
    counts = jax.ops.segment_sum(jnp.ones_like(batch), batch, num_segments=G)
    k1 = jnp.ceil(RATIO * counts.astype(jnp.float32)).astype(jnp.int32)
    k2 = jnp.ceil(RATIO * k1.astype(jnp.float32)).astype(jnp.int32)

    h = jax.nn.relu(_graph_conv1(x, src, dst, W1_rel, b1_rel, W1_root))
    s1 = jnp.tanh((h @ pw1) / jnp.linalg.norm(pw1))
    all_mask = jnp.ones((h.shape[0],), dtype=bool)
    sel1 = _topk_select(s1, batch, all_mask, k1, G)
    hp1 = h * s1[:, None] * sel1[:, None].astype(h.dtype)
    x1 = _global_pools(hp1, batch, sel1, k1, G)

    h2 = jax.nn.relu(_graph_conv2(hp1, sel1, src, dst, W2_rel, b2_rel, W2_root))
    s2 = jnp.tanh((h2 @ pw2) / jnp.linalg.norm(pw2))
    sel2 = _topk_select(s2, batch, sel1, k2, G)
    hp2 = h2 * s2[:, None] * sel2[:, None].astype(h2.dtype)
    x2 = _global_pools(hp2, batch, sel2, k2, G)

    z = x1 + x2
    return _mlp_head(z, W_l1, b_l1, W_l2, b_l2, W_l3, b_l3)


# speedup vs baseline: 22.5070x; 1.0027x over previous
import functools
import jax
import jax.numpy as jnp
from jax import lax
from jax.experimental import pallas as pl
from jax.experimental.pallas import tpu as pltpu
from jax.experimental.pallas import tpu_sc as plsc

NUM_GRAPHS = 128
RATIO = 0.8

N_NODES = 100000
N_EDGES = 3200000
NC = 2
NS = 16
EDGES_PER_WORKER = N_EDGES // (NC * NS)
ROWS_PER_TILE = 6256
N_PAD = ROWS_PER_TILE * NS


def _edge_win(d):
    return 1000 if d else 2000


ZCHUNK = 272
N_ZCOPY = ROWS_PER_TILE // ZCHUNK


def _make_edge_agg_body(d):
    zrows = ZCHUNK * (d if d else 1) // 16

    ew = _edge_win(d)
    n_wins = EDGES_PER_WORKER // ew

    def body(table_hbm, src_hbm, dst_hbm, out_hbm,
             agg_sh, src_v, dst_v, rows_v, zbuf, gsem):
        c = lax.axis_index("c")
        s = lax.axis_index("s")
        wid = c * NS + s

        def zstore(i, carry):
            if d:
                zbuf[i, :] = jnp.zeros((16,), jnp.float32)
            else:
                zbuf[pl.ds(i * 16, 16)] = jnp.zeros((16,), jnp.float32)
            return carry

        lax.fori_loop(0, (ZCHUNK * d // 16) if d else (ZCHUNK // 16), zstore, 0)
        for j in range(N_ZCOPY):
            pltpu.sync_copy(
                zbuf, agg_sh.at[pl.ds(s * ROWS_PER_TILE + j * ZCHUNK, ZCHUNK)])
        plsc.subcore_barrier()

        def win(w, carry):
            base = wid * EDGES_PER_WORKER + w * ew
            pltpu.sync_copy(src_hbm.at[pl.ds(base, ew)], src_v)
            pltpu.async_copy(table_hbm.at[src_v], rows_v, gsem).wait()
            pltpu.sync_copy(dst_hbm.at[pl.ds(base, ew)], dst_v)
            pltpu.sync_copy(rows_v, agg_sh.at[dst_v], add=True)
            return carry

        lax.fori_loop(0, n_wins, win, 0)
        plsc.subcore_barrier()
        for j in range(N_ZCOPY):
            off = s * ROWS_PER_TILE + j * ZCHUNK
            pltpu.sync_copy(agg_sh.at[pl.ds(off, ZCHUNK)], zbuf)
            pltpu.sync_copy(zbuf, out_hbm.at[pl.ds(c * N_PAD + off, ZCHUNK)])

    return body


def _edge_agg_rows(table, src, dst, d):
    mesh = plsc.VectorSubcoreMesh(core_axis_name="c", subcore_axis_name="s")
    if d:
        row = (d,)
    else:
        row = ()
    fn = pl.kernel(
        _make_edge_agg_body(d),
        out_type=jax.ShapeDtypeStruct((NC * N_PAD,) + row, jnp.float32),
        mesh=mesh,
        compiler_params=pltpu.CompilerParams(use_tc_tiling_on_sc=False),
        scratch_types=[
            pltpu.VMEM_SHARED((N_PAD,) + row, jnp.float32),
            pltpu.VMEM((_edge_win(d),), jnp.int32),
            pltpu.VMEM((_edge_win(d),), jnp.int32),
            pltpu.VMEM((_edge_win(d),) + row, jnp.float32),
            pltpu.VMEM((ZCHUNK,) + row, jnp.float32),
            pltpu.SemaphoreType.DMA,
        ],
    )
    parts = fn(table, src, dst)
    return parts[:N_NODES] + parts[N_PAD:N_PAD + N_NODES]


def _head_body(z_ref, w1_ref, b1_ref, w2_ref, b2_ref, w3_ref, b3_ref, o_ref):
    z = z_ref[...]
    h = jnp.maximum(jnp.dot(z, w1_ref[...], preferred_element_type=jnp.float32) + b1_ref[...], 0.0)
    h = jnp.maximum(jnp.dot(h, w2_ref[...], preferred_element_type=jnp.float32) + b2_ref[...], 0.0)
    o = jnp.dot(h, w3_ref[...], preferred_element_type=jnp.float32) + b3_ref[...]
    m = jnp.max(o, axis=-1, keepdims=True)
    lse = m + jnp.log(jnp.sum(jnp.exp(o - m), axis=-1, keepdims=True))
    o_ref[...] = o - lse


def _mlp_head(z, W_l1, b_l1, W_l2, b_l2, W_l3, b_l3):
    return pl.pallas_call(
        _head_body,
        out_shape=jax.ShapeDtypeStruct((z.shape[0], W_l3.shape[0]), jnp.float32),
    )(z, W_l1.T, b_l1[None, :], W_l2.T, b_l2[None, :], W_l3.T, b_l3[None, :])


def _topk_select(score, batch, node_mask, k_per_graph, num_graphs):
    s = jnp.where(node_mask, score, -jnp.inf)
    sort_key = batch.astype(jnp.float32) * 4.0 - s
    order = jnp.argsort(sort_key)
    active = jax.ops.segment_sum(node_mask.astype(jnp.int32), batch,
                                 num_segments=num_graphs)
    starts = jnp.cumsum(active) - active
    nn = score.shape[0]
    rank = jnp.zeros((nn,), jnp.int32).at[order].set(
        jnp.arange(nn, dtype=jnp.int32) - starts[batch[order]])
    return (rank < k_per_graph[batch]) & node_mask


def _pools_jnp(xv, batch, selb, kf):
    mx = jax.ops.segment_max(jnp.where(selb[:, None], xv, -jnp.inf), batch,
                             num_segments=NUM_GRAPHS)
    mx = jnp.where(jnp.isfinite(mx), mx, 0.0)
    sm = jax.ops.segment_sum(xv, batch, num_segments=NUM_GRAPHS)
    return jnp.concatenate([mx, sm / jnp.maximum(kf, 1.0)[:, None]], axis=1)


def kernel(x, edge_index, batch, W1_rel, b1_rel, W1_root, pw1, W2_rel, b2_rel, W2_root, pw2, W_l1, b_l1, W_l2, b_l2, W_l3, b_l3):
    src, dst = edge_index[0], edge_index[1]
    G = NUM_GRAPHS
    n = x.shape[0]

    agg1 = _edge_agg_rows(x[:, 0], src, dst, 0)[:, None]
    h = jax.nn.relu(agg1 @ W1_rel.T + b1_rel[None, :] + x @ W1_root.T)
    s1 = jnp.tanh((h @ pw1) / jnp.linalg.norm(pw1))

    counts = jax.ops.segment_sum(jnp.ones_like(batch), batch, num_segments=G)
    k1 = jnp.ceil(RATIO * counts.astype(jnp.float32)).astype(jnp.int32)
    sel1 = _topk_select(s1, batch, jnp.ones((n,), bool), k1, G)
    sel1f = sel1.astype(jnp.float32)
    hp1 = h * (s1 * sel1f)[:, None]
    x1 = _pools_jnp(hp1, batch, sel1, k1.astype(jnp.float32))

    agg2 = _edge_agg_rows(hp1, src, dst, 16) * sel1f[:, None]
    h2 = jax.nn.relu(agg2 @ W2_rel.T + b2_rel[None, :] + hp1 @ W2_root.T)
    s2 = jnp.tanh((h2 @ pw2) / jnp.linalg.norm(pw2))

    k2 = jnp.ceil(RATIO * k1.astype(jnp.float32)).astype(jnp.int32)
    sel2 = _topk_select(s2, batch, sel1, k2, G)
    hp2 = h2 * (s2 * sel2.astype(jnp.float32))[:, None]
    x2 = _pools_jnp(hp2, batch, sel2, k2.astype(jnp.float32))

    z = x1 + x2
    return _mlp_head(z, W_l1, b_l1, W_l2, b_l2, W_l3, b_l3)
